# Initial kernel scaffold; baseline (speedup 1.0000x reference)
import functools

import jax
import jax.numpy as jnp
from jax import lax
from jax.experimental import pallas as pl
from jax.experimental.pallas import tpu as pltpu
from jax.experimental.pallas import tpu_sc as plsc

N = 10000
E = 160000
H = 128
NB = 8
SUB = H // NB
R = 8

NC = 2
NS = 16
EPT = E // NS
C = 80
NCH = EPT // C
NSEG = N * R
RPT = NSEG // NS
ZC = 1000
PASSES = NB // NC

_mesh = plsc.VectorSubcoreMesh(
    core_axis_name="c", subcore_axis_name="s", num_cores=NC, num_subcores=NS
)


@functools.partial(
    pl.kernel,
    out_type=jax.ShapeDtypeStruct((NB, NSEG, SUB), jnp.float32),
    mesh=_mesh,
    scratch_types=[
        pltpu.VMEM_SHARED((NSEG, SUB), jnp.float32),
        pltpu.VMEM((EPT,), jnp.int32),
        pltpu.VMEM((EPT,), jnp.int32),
        pltpu.VMEM((EPT,), jnp.int32),
        pltpu.VMEM((EPT,), jnp.float32),
        pltpu.VMEM((NCH, C), jnp.int32),
        pltpu.VMEM((NCH, C), jnp.int32),
        pltpu.VMEM((C, SUB), jnp.float32),
        pltpu.VMEM((ZC, SUB), jnp.float32),
        pltpu.SemaphoreType.DMA,
    ],
)
def _sc_accumulate(xsl, srch, dsth, eth, normh, a_out,
                   acc, srcv, dstv, etv, normv, idx2, seg2, rows, zbuf, sem):
    c = lax.axis_index("c")
    s = lax.axis_index("s")
    e0 = s * EPT

    pltpu.sync_copy(srch.at[pl.ds(e0, EPT)], srcv)
    pltpu.sync_copy(dsth.at[pl.ds(e0, EPT)], dstv)
    pltpu.sync_copy(eth.at[pl.ds(e0, EPT)], etv)
    pltpu.sync_copy(normh.at[pl.ds(e0, EPT)], normv)

    def _zb(i, carry):
        zbuf[i] = jnp.zeros((SUB,), jnp.float32)
        return carry
    lax.fori_loop(0, ZC, _zb, 0)

    def _mkseg(j, carry):
        for g in range(C // SUB):
            off = j * C + g * SUB
            seg2[j, pl.ds(g * SUB, SUB)] = (
                dstv[pl.ds(off, SUB)] * R + etv[pl.ds(off, SUB)]
            )
        return carry
    lax.fori_loop(0, NCH, _mkseg, 0)

    for p in range(PASSES):
        b = c * PASSES + p

        for k in range(RPT // ZC):
            pltpu.sync_copy(zbuf, acc.at[pl.ds(s * RPT + k * ZC, ZC)])

        def _mkidx(j, carry):
            for g in range(C // SUB):
                off = j * C + g * SUB
                idx2[j, pl.ds(g * SUB, SUB)] = srcv[pl.ds(off, SUB)] + b * N
            return carry
        lax.fori_loop(0, NCH, _mkidx, 0)

        plsc.subcore_barrier()

        def _chunk(j, carry):
            pltpu.async_copy(xsl.at[idx2.at[j]], rows, sem).wait()
            for e in range(C):
                rows[e] = rows[e] * normv[j * C + e]
            pltpu.sync_copy(rows, acc.at[seg2.at[j]], add=True)
            return carry
        lax.fori_loop(0, NCH, _chunk, 0)

        plsc.subcore_barrier()

        pltpu.sync_copy(acc.at[pl.ds(s * RPT, RPT)],
                        a_out.at[b, pl.ds(s * RPT, RPT)])

        plsc.subcore_barrier()


NT = 400


def _tc_body(a_ref, x_ref, wpad_ref, lw_ref, bias_ref, o_ref):
    acc = jnp.dot(x_ref[...], lw_ref[...], preferred_element_type=jnp.float32)
    acc = acc + bias_ref[...]
    for b in range(NB):
        acc = acc + jnp.dot(a_ref[b], wpad_ref[b],
                            preferred_element_type=jnp.float32)
    o_ref[...] = acc


_tc_combine = pl.pallas_call(
    _tc_body,
    grid=(N // NT,),
    in_specs=[
        pl.BlockSpec((NB, NT, H), lambda i: (0, i, 0)),
        pl.BlockSpec((NT, H), lambda i: (i, 0)),
        pl.BlockSpec((NB, H, H), lambda i: (0, 0, 0)),
        pl.BlockSpec((H, H), lambda i: (0, 0)),
        pl.BlockSpec((1, H), lambda i: (0, 0)),
    ],
    out_specs=pl.BlockSpec((NT, H), lambda i: (i, 0)),
    out_shape=jax.ShapeDtypeStruct((N, H), jnp.float32),
)


def kernel(x, edge_index, etype, norm, weight, loop_weight, h_bias):
    src = edge_index[0]
    dst = edge_index[1]
    normf = norm.reshape(E)
    xsl = x.reshape(N, NB, SUB).transpose(1, 0, 2).reshape(NB * N, SUB)

    a = _sc_accumulate(xsl, src, dst, etype, normf)
    a3 = a.reshape(NB, N, H)

    wsm = weight.transpose(1, 0, 2, 3).reshape(NB, H, SUB)
    bidx = jnp.arange(NB)[:, None, None]
    kidx = jnp.arange(H)[None, :, None]
    oidx = (jnp.arange(NB) * SUB)[:, None, None] + jnp.arange(SUB)[None, None, :]
    wpad = jnp.zeros((NB, H, H), jnp.float32).at[bidx, kidx, oidx].set(wsm)

    return _tc_combine(a3, x, wpad, loop_weight, h_bias.reshape(1, H))


# speedup vs baseline: 3.7158x; 3.7158x over previous
import functools

import jax
import jax.numpy as jnp
from jax import lax
from jax.experimental import pallas as pl
from jax.experimental.pallas import tpu as pltpu
from jax.experimental.pallas import tpu_sc as plsc

N = 10000
E = 160000
H = 128
NB = 8
SUB = H // NB
R = 8

NC = 2
NS = 16
NW = NC * NS
EPT = E // NW
C = 64
NCH = -(-EPT // C)
EPT_PAD = NCH * C
ZR = 40
ROWS_PER_WR = 1000

_mesh = plsc.VectorSubcoreMesh(
    core_axis_name="c", subcore_axis_name="s", num_cores=NC, num_subcores=NS
)


@functools.partial(
    pl.kernel,
    out_type=jax.ShapeDtypeStruct((NC, N, H), jnp.float32),
    mesh=_mesh,
    scratch_types=[
        pltpu.VMEM_SHARED((N, H), jnp.float32),
        pltpu.VMEM((EPT_PAD,), jnp.int32),
        pltpu.VMEM((EPT_PAD,), jnp.int32),
        pltpu.VMEM((EPT_PAD,), jnp.int32),
        pltpu.VMEM((EPT_PAD,), jnp.float32),
        pltpu.VMEM((1, C), jnp.int32),
        pltpu.VMEM((1, C), jnp.int32),
        pltpu.VMEM((C, H), jnp.float32),
        pltpu.VMEM((ZR, H), jnp.float32),
        pltpu.SemaphoreType.DMA,
    ],
)
def _sc_edge_pass(xt, srch, dsth, eth, normh, a_out,
                  acc, srcv, dstv, etv, normv, idx1, seg1, rows, zbuf, sem):
    c = lax.axis_index("c")
    s = lax.axis_index("s")
    t = c * NS + s
    e0 = t * EPT

    pltpu.sync_copy(srch.at[pl.ds(e0, EPT)], srcv.at[pl.ds(0, EPT)])
    pltpu.sync_copy(dsth.at[pl.ds(e0, EPT)], dstv.at[pl.ds(0, EPT)])
    pltpu.sync_copy(eth.at[pl.ds(e0, EPT)], etv.at[pl.ds(0, EPT)])
    pltpu.sync_copy(normh.at[pl.ds(e0, EPT)], normv.at[pl.ds(0, EPT)])

    def _zb(i, carry):
        for q in range(H // SUB):
            zbuf[i, pl.ds(q * SUB, SUB)] = jnp.zeros((SUB,), jnp.float32)
        return carry
    lax.fori_loop(0, ZR, _zb, 0)

    lanes = lax.iota(jnp.int32, SUB)

    @pl.when(s < N // ROWS_PER_WR)
    def _clear():
        for k in range(ROWS_PER_WR // ZR):
            pltpu.sync_copy(zbuf, acc.at[pl.ds(s * ROWS_PER_WR + k * ZR, ZR)])

    plsc.subcore_barrier()

    def _chunk(j, carry):
        for g in range(C // SUB):
            off = j * C + g * SUB
            ok = lanes < (EPT - off)
            sv = srcv[pl.ds(off, SUB)]
            ev = etv[pl.ds(off, SUB)]
            dv = dstv[pl.ds(off, SUB)]
            idx1[0, pl.ds(g * SUB, SUB)] = jnp.where(ok, ev * N + sv, 0)
            seg1[0, pl.ds(g * SUB, SUB)] = jnp.where(ok, dv, 0)
        pltpu.async_copy(xt.at[idx1.at[0]], rows, sem).wait()
        for g in range(C // SUB):
            off = j * C + g * SUB
            ok = lanes < (EPT - off)
            nv = jnp.where(ok, normv[pl.ds(off, SUB)], 0.0)
            for e in range(SUB):
                r = g * SUB + e
                sc = nv[e]
                for q in range(H // SUB):
                    rows[r, pl.ds(q * SUB, SUB)] = (
                        rows[r, pl.ds(q * SUB, SUB)] * sc
                    )
        pltpu.sync_copy(rows, acc.at[seg1.at[0]], add=True)
        return carry
    lax.fori_loop(0, NCH, _chunk, 0)

    plsc.subcore_barrier()

    @pl.when(s < N // ROWS_PER_WR)
    def _writeout():
        pltpu.sync_copy(acc.at[pl.ds(s * ROWS_PER_WR, ROWS_PER_WR)],
                        a_out.at[c, pl.ds(s * ROWS_PER_WR, ROWS_PER_WR)])


NT = 400


def _tc_pre_body(x_ref, w_ref, o_ref):
    o_ref[...] = jnp.dot(x_ref[...], w_ref[0],
                         preferred_element_type=jnp.float32)


_tc_pretransform = pl.pallas_call(
    _tc_pre_body,
    grid=(R, N // NT),
    in_specs=[
        pl.BlockSpec((NT, H), lambda r, i: (i, 0)),
        pl.BlockSpec((1, H, H), lambda r, i: (r, 0, 0)),
    ],
    out_specs=pl.BlockSpec((NT, H), lambda r, i: (r * (N // NT) + i, 0)),
    out_shape=jax.ShapeDtypeStruct((R * N, H), jnp.float32),
)


def _tc_comb_body(a_ref, x_ref, lw_ref, bias_ref, o_ref):
    acc = jnp.dot(x_ref[...], lw_ref[...], preferred_element_type=jnp.float32)
    o_ref[...] = acc + bias_ref[...] + a_ref[0] + a_ref[1]


_tc_combine = pl.pallas_call(
    _tc_comb_body,
    grid=(N // NT,),
    in_specs=[
        pl.BlockSpec((NC, NT, H), lambda i: (0, i, 0)),
        pl.BlockSpec((NT, H), lambda i: (i, 0)),
        pl.BlockSpec((H, H), lambda i: (0, 0)),
        pl.BlockSpec((1, H), lambda i: (0, 0)),
    ],
    out_specs=pl.BlockSpec((NT, H), lambda i: (i, 0)),
    out_shape=jax.ShapeDtypeStruct((N, H), jnp.float32),
)


def kernel(x, edge_index, etype, norm, weight, loop_weight, h_bias):
    src = edge_index[0]
    dst = edge_index[1]
    normf = norm.reshape(E)

    kidx = (jnp.arange(NB) * SUB)[:, None, None] + jnp.arange(SUB)[None, :, None]
    oidx = (jnp.arange(NB) * SUB)[:, None, None] + jnp.arange(SUB)[None, None, :]
    wbd = jnp.zeros((R, H, H), jnp.float32).at[:, kidx, oidx].set(
        weight.reshape(R, NB, SUB, SUB)
    )

    xt = _tc_pretransform(x, wbd)
    a = _sc_edge_pass(xt, src, dst, etype, normf)
    return _tc_combine(a, x, loop_weight, h_bias.reshape(1, H))
